# baseline (speedup 1.0000x reference)
import jax
import jax.numpy as jnp
import numpy as np
from jax import lax
from jax.experimental import pallas as pl
from jax.experimental.pallas import tpu as pltpu
from jax.experimental.pallas import tpu_sc as plsc

B = 4
N = 16800
L = 16
NC = 2
NS = 16
VAR0 = np.float32(0.1)
NEG_INF = np.float32(-np.inf)
IMAX = np.int32(2**31 - 1)

_MESH = plsc.VectorSubcoreMesh(
    core_axis_name="c", subcore_axis_name="s", num_cores=NC, num_subcores=NS
)


def _sc_body(conf_hbm, landms_hbm, priors_hbm, out_hbm, sbuf, lrow, prow, obuf):
    c = lax.axis_index("c")
    s = lax.axis_index("s")
    b = s * NC + c

    @pl.when(b < B)
    def _():
        pltpu.sync_copy(conf_hbm.at[b], sbuf)

        lane = lax.iota(jnp.int32, L)

        def step(j, carry):
            run_max, run_idx = carry
            n = j * L + lane
            v = plsc.load_gather(sbuf, [n * 2 + 1])
            v = jnp.where(v > 0.0, v, NEG_INF)
            upd = v > run_max
            return jnp.where(upd, v, run_max), jnp.where(upd, n, run_idx)

        run_max = jnp.full((L,), NEG_INF, jnp.float32)
        run_idx = jnp.zeros((L,), jnp.int32)
        run_max, run_idx = lax.fori_loop(0, N // L, step, (run_max, run_idx))

        best = jnp.max(run_max, axis=0)
        cand = jnp.where(run_max == best, run_idx, IMAX)
        idx = jnp.min(cand, axis=0)

        pltpu.sync_copy(landms_hbm.at[b, pl.ds(idx, 1), :], lrow)
        pltpu.sync_copy(priors_hbm.at[pl.ds(idx, 1), :], prow)

        zeros = jnp.zeros((L,), jnp.int32)
        par = lane & 1
        lvec = plsc.load_gather(lrow, [zeros, jnp.minimum(lane, 9)])
        pxy = plsc.load_gather(prow, [zeros, par])
        pwh = plsc.load_gather(prow, [zeros, par + 2])

        obuf[...] = pxy + lvec * VAR0 * pwh
        pltpu.sync_copy(obuf, out_hbm.at[b])


_sc_call = pl.kernel(
    _sc_body,
    out_type=jax.ShapeDtypeStruct((B, L), jnp.float32),
    mesh=_MESH,
    compiler_params=pltpu.CompilerParams(needs_layout_passes=False),
    scratch_types=[
        pltpu.VMEM((2 * N,), jnp.float32),
        pltpu.VMEM((1, 10), jnp.float32),
        pltpu.VMEM((1, 4), jnp.float32),
        pltpu.VMEM((L,), jnp.float32),
    ],
)


def kernel(loc, conf, landms, priors):
    del loc
    out = _sc_call(conf.reshape(B, 2 * N), landms, priors)
    return out[:, :10]
